# baseline (speedup 1.0000x reference)
import functools

import jax
import jax.numpy as jnp
from jax import lax
from jax.experimental import pallas as pl
from jax.experimental.pallas import tpu as pltpu
from jax.experimental.pallas import tpu_sc as plsc

_N_WORKERS = 32
_CHUNK = 128
_LANES = 16


@functools.partial(jax.jit, static_argnums=(6, 7))
def _mf_call(user_idx, item_idx, user_factors, item_factors,
             user_biases, item_biases, b_per_w, d):
    batch = user_idx.shape[0] * user_idx.shape[1] * user_idx.shape[2]
    n_chunks = b_per_w // _CHUNK
    n_groups = b_per_w // _LANES
    mesh = plsc.VectorSubcoreMesh(core_axis_name="c", subcore_axis_name="s")

    @functools.partial(
        pl.kernel,
        out_type=jax.ShapeDtypeStruct((batch,), jnp.float32),
        mesh=mesh,
        compiler_params=pltpu.CompilerParams(
            needs_layout_passes=False, use_tc_tiling_on_sc=False),
        scratch_types=[
            pltpu.VMEM((n_chunks, _CHUNK), jnp.int32),
            pltpu.VMEM((n_chunks, _CHUNK), jnp.int32),
            pltpu.VMEM((b_per_w, d), jnp.float32),
            pltpu.VMEM((b_per_w, d), jnp.float32),
            pltpu.VMEM((b_per_w,), jnp.float32),
            pltpu.VMEM((b_per_w,), jnp.float32),
            pltpu.VMEM((b_per_w,), jnp.float32),
            pltpu.SemaphoreType.DMA,
        ],
    )
    def k(uidx_hbm, iidx_hbm, uf_hbm, if_hbm, ub_hbm, ib_hbm, out_hbm,
          uidx_v, iidx_v, urows_v, irows_v, ub_v, ib_v, out_v, sem):
        wid = lax.axis_index("s") * 2 + lax.axis_index("c")
        base = wid * b_per_w
        pltpu.sync_copy(uidx_hbm.at[wid], uidx_v)
        pltpu.sync_copy(iidx_hbm.at[wid], iidx_v)
        copies = []
        for j in range(n_chunks):
            dst = pl.ds(j * _CHUNK, _CHUNK)
            copies.append(pltpu.async_copy(
                uf_hbm.at[uidx_v.at[j]], urows_v.at[dst], sem))
            copies.append(pltpu.async_copy(
                if_hbm.at[iidx_v.at[j]], irows_v.at[dst], sem))
            copies.append(pltpu.async_copy(
                ub_hbm.at[uidx_v.at[j]], ub_v.at[dst], sem))
            copies.append(pltpu.async_copy(
                ib_hbm.at[iidx_v.at[j]], ib_v.at[dst], sem))
        for c in copies:
            c.wait()

        def group_body(g, _):
            sl = pl.ds(g * _LANES, _LANES)
            rows = lax.iota(jnp.int32, _LANES) + g * _LANES
            acc = ub_v[sl] + ib_v[sl]
            for dd in range(d):
                cols = jnp.full((_LANES,), dd, jnp.int32)
                uf = plsc.load_gather(urows_v, [rows, cols])
                vf = plsc.load_gather(irows_v, [rows, cols])
                acc = acc + uf * vf
            out_v[sl] = acc
            return 0

        lax.fori_loop(0, n_groups, group_body, 0)
        pltpu.sync_copy(out_v, out_hbm.at[pl.ds(base, b_per_w)])

    return k(user_idx, item_idx, user_factors, item_factors,
             user_biases, item_biases)


def kernel(user_item_tuple, user_factors, item_factors, user_biases, item_biases):
    batch = user_item_tuple.shape[0]
    d = user_factors.shape[1]
    b_per_w = batch // _N_WORKERS
    n_chunks = b_per_w // _CHUNK
    user_idx = user_item_tuple[:, 0].reshape(_N_WORKERS, n_chunks, _CHUNK)
    item_idx = user_item_tuple[:, 1].reshape(_N_WORKERS, n_chunks, _CHUNK)
    return _mf_call(user_idx, item_idx, user_factors, item_factors,
                    user_biases.reshape(-1), item_biases.reshape(-1),
                    b_per_w, d)
